# baseline (speedup 1.0000x reference)
import functools

import jax
import jax.numpy as jnp
import numpy as np
from jax.experimental import pallas as pl

S, H = 2048, 1024
NH, DH = 16, 64
FF = 2048
NUM_REC = 3
NUM_EXPERTS = 2 ** NUM_REC
ROW_BLK = 512


def _pcall(body, **kw):
    return pl.pallas_call(body, **kw)


def _router_body(x_ref, ek_ref, kmask_ref, sw_ref):
    x = x_ref[...]
    scores = jax.lax.dot_general(
        x, ek_ref[...], (((1,), (1,)), ((), ())),
        preferred_element_type=jnp.float32)
    m = jnp.max(scores, axis=-1, keepdims=True)
    w = 1.0 / jnp.sum(jnp.exp(scores - m), axis=-1, keepdims=True)
    chosen = jnp.argmax(scores, axis=-1).astype(jnp.int32)
    bits = jax.lax.broadcasted_iota(jnp.int32, (S, NUM_EXPERTS), 1)
    active = ((chosen[:, None] >> bits) & 1).astype(jnp.float32)
    sw_ref[...] = active * w
    kmask_ref[...] = active.T


def _qkv_body(x_ref, ek_ref, norm_ref, pos_ref, wq_ref, wk_ref, wv_ref,
              q_ref, k_ref, v_ref, xpre_ref):
    x = x_ref[...] + ek_ref[...]
    xpre_ref[...] = x
    h = x * jax.lax.rsqrt(jnp.mean(x * x, axis=-1, keepdims=True) + 1e-6)
    h = (h * norm_ref[...]).astype(jnp.bfloat16)
    q = jnp.dot(h, wq_ref[...], preferred_element_type=jnp.float32)
    k = jnp.dot(h, wk_ref[...], preferred_element_type=jnp.float32)
    v = jnp.dot(h, wv_ref[...], preferred_element_type=jnp.float32)

    posf = pos_ref[...].astype(jnp.float32)
    col = jax.lax.broadcasted_iota(jnp.int32, (1, H), 1)
    offs = col % DH
    f = (offs % (DH // 2)).astype(jnp.float32)
    inv = jnp.exp(f * (-np.log(10000.0) / (DH // 2)))
    ang = posf * inv
    cosf = jnp.cos(ang)
    sinf = jnp.sin(ang)
    first_half = offs < (DH // 2)

    def rope(t):
        rot_m = jnp.concatenate([t[:, DH // 2:], t[:, :DH // 2]], axis=1)
        rot_p = jnp.concatenate([t[:, -(DH // 2):], t[:, :-(DH // 2)]], axis=1)
        rot = jnp.where(first_half, -rot_m, rot_p)
        return t * cosf + rot * sinf

    q_ref[...] = rope(q)
    k_ref[...] = rope(k)
    v_ref[...] = v


def _attn_body(q_ref, k_ref, v_ref, m_ref, o_ref):
    mask = m_ref[...] > 0.5
    for sub in range(2):
        sl = slice(sub * DH, (sub + 1) * DH)
        q = q_ref[:, sl]
        att = jax.lax.dot_general(
            q, k_ref[:, sl], (((1,), (1,)), ((), ())),
            preferred_element_type=jnp.float32) * (1.0 / np.sqrt(DH))
        att = jnp.where(mask, att, -1e30)
        mx = jnp.max(att, axis=-1, keepdims=True)
        e = jnp.exp(att - mx)
        p = e / jnp.sum(e, axis=-1, keepdims=True)
        o_ref[:, sl] = jnp.dot(p, v_ref[:, sl],
                               preferred_element_type=jnp.float32)


def _post_body(final_ref, xpre_ref, o_ref, sw_ref, wo_ref, norm_ref,
               wg_ref, wu_ref, wd_ref, out_ref):
    x = xpre_ref[...] + jnp.dot(o_ref[...].astype(jnp.bfloat16), wo_ref[...],
                                preferred_element_type=jnp.float32)
    h2 = x * jax.lax.rsqrt(jnp.mean(x * x, axis=-1, keepdims=True) + 1e-6)
    h2 = (h2 * norm_ref[...]).astype(jnp.bfloat16)
    g = jnp.dot(h2, wg_ref[...], preferred_element_type=jnp.float32)
    u = jnp.dot(h2, wu_ref[...], preferred_element_type=jnp.float32)
    act = ((g * jax.lax.logistic(g)) * u).astype(jnp.bfloat16)
    x = x + jnp.dot(act, wd_ref[...], preferred_element_type=jnp.float32)
    out_ref[...] = final_ref[...] + x * sw_ref[...]


def kernel(hidden_states, position_ids, expert_keys, params):
    Bb, Ss, Hh = hidden_states.shape
    flat = hidden_states.reshape(Ss, Hh)
    pos = position_ids.reshape(Ss, 1).astype(jnp.int32)

    kmask, sw = _pcall(
        _router_body,
        out_shape=(
            jax.ShapeDtypeStruct((NUM_EXPERTS, S), jnp.float32),
            jax.ShapeDtypeStruct((S, NUM_EXPERTS), jnp.float32),
        ),
    )(flat, expert_keys)

    nrow = S // ROW_BLK
    final = flat
    for d in range(NUM_REC):
        p = params[d]
        ek_row = expert_keys[1 << d][None, :]
        wq, wk, wv, wo, wg, wu, wd = (
            p[n].astype(jnp.bfloat16)
            for n in ("Wq", "Wk", "Wv", "Wo", "Wg", "Wu", "Wd"))

        q, k, v, xpre = _pcall(
            _qkv_body,
            grid=(nrow,),
            in_specs=[
                pl.BlockSpec((ROW_BLK, H), lambda i: (i, 0)),
                pl.BlockSpec((1, H), lambda i: (0, 0)),
                pl.BlockSpec((1, H), lambda i: (0, 0)),
                pl.BlockSpec((ROW_BLK, 1), lambda i: (i, 0)),
                pl.BlockSpec((H, H), lambda i: (0, 0)),
                pl.BlockSpec((H, H), lambda i: (0, 0)),
                pl.BlockSpec((H, H), lambda i: (0, 0)),
            ],
            out_specs=[
                pl.BlockSpec((ROW_BLK, H), lambda i: (i, 0)),
                pl.BlockSpec((ROW_BLK, H), lambda i: (i, 0)),
                pl.BlockSpec((ROW_BLK, H), lambda i: (i, 0)),
                pl.BlockSpec((ROW_BLK, H), lambda i: (i, 0)),
            ],
            out_shape=[jax.ShapeDtypeStruct((S, H), jnp.float32)] * 4,
        )(final, ek_row, p["attn_norm"][None, :], pos, wq, wk, wv)

        o = _pcall(
            _attn_body,
            grid=(NH // 2,),
            in_specs=[
                pl.BlockSpec((S, 2 * DH), lambda h: (0, h)),
                pl.BlockSpec((S, 2 * DH), lambda h: (0, h)),
                pl.BlockSpec((S, 2 * DH), lambda h: (0, h)),
                pl.BlockSpec((1, S), lambda h: (0, 0)),
            ],
            out_specs=pl.BlockSpec((S, 2 * DH), lambda h: (0, h)),
            out_shape=jax.ShapeDtypeStruct((S, H), jnp.float32),
        )(q, k, v, kmask[d][None, :])

        final = _pcall(
            _post_body,
            grid=(nrow,),
            in_specs=[
                pl.BlockSpec((ROW_BLK, H), lambda i: (i, 0)),
                pl.BlockSpec((ROW_BLK, H), lambda i: (i, 0)),
                pl.BlockSpec((ROW_BLK, H), lambda i: (i, 0)),
                pl.BlockSpec((ROW_BLK, 1), lambda i: (i, 0)),
                pl.BlockSpec((H, H), lambda i: (0, 0)),
                pl.BlockSpec((1, H), lambda i: (0, 0)),
                pl.BlockSpec((H, FF), lambda i: (0, 0)),
                pl.BlockSpec((H, FF), lambda i: (0, 0)),
                pl.BlockSpec((FF, H), lambda i: (0, 0)),
            ],
            out_specs=pl.BlockSpec((ROW_BLK, H), lambda i: (i, 0)),
            out_shape=jax.ShapeDtypeStruct((S, H), jnp.float32),
        )(final, xpre, o, sw[:, d:d + 1], wo, p["mlp_norm"][None, :],
          wg, wu, wd)

    return final.reshape(Bb, Ss, Hh)


# speedup vs baseline: 1.0739x; 1.0739x over previous
import functools

import jax
import jax.numpy as jnp
import numpy as np
from jax.experimental import pallas as pl

S, H = 2048, 1024
NH, DH = 16, 64
FF = 2048
NUM_REC = 3
NUM_EXPERTS = 2 ** NUM_REC
ROW_BLK = 512


def _pcall(body, **kw):
    return pl.pallas_call(body, **kw)


def _router_body(x_ref, ek_ref, kmask_ref, sw_ref):
    x = x_ref[...]
    scores = jax.lax.dot_general(
        x, ek_ref[...], (((1,), (1,)), ((), ())),
        preferred_element_type=jnp.float32)
    m = jnp.max(scores, axis=-1, keepdims=True)
    w = 1.0 / jnp.sum(jnp.exp(scores - m), axis=-1, keepdims=True)
    chosen = jnp.argmax(scores, axis=-1).astype(jnp.int32)
    bits = jax.lax.broadcasted_iota(jnp.int32, (S, NUM_EXPERTS), 1)
    active = ((chosen[:, None] >> bits) & 1).astype(jnp.float32)
    sw_ref[...] = active * w
    kmask_ref[...] = active.T


def _qkv_body(x_ref, ek_ref, norm_ref, pos_ref, wq_ref, wk_ref, wv_ref,
              q_ref, k_ref, v_ref):
    x = x_ref[...] + ek_ref[...]
    h = x * jax.lax.rsqrt(jnp.mean(x * x, axis=-1, keepdims=True) + 1e-6)
    h = (h * norm_ref[...]).astype(jnp.bfloat16)
    q = jnp.dot(h, wq_ref[...], preferred_element_type=jnp.float32)
    k = jnp.dot(h, wk_ref[...], preferred_element_type=jnp.float32)
    v = jnp.dot(h, wv_ref[...], preferred_element_type=jnp.float32)

    posf = pos_ref[...].astype(jnp.float32)
    col = jax.lax.broadcasted_iota(jnp.int32, (1, H), 1)
    offs = col % DH
    f = (offs % (DH // 2)).astype(jnp.float32)
    inv = jnp.exp(f * (-np.log(10000.0) / (DH // 2)))
    ang = posf * inv
    cosf = jnp.cos(ang)
    sinf = jnp.sin(ang)
    first_half = offs < (DH // 2)

    def rope(t):
        rot_m = jnp.concatenate([t[:, DH // 2:], t[:, :DH // 2]], axis=1)
        rot_p = jnp.concatenate([t[:, -(DH // 2):], t[:, :-(DH // 2)]], axis=1)
        rot = jnp.where(first_half, -rot_m, rot_p)
        return t * cosf + rot * sinf

    q_ref[...] = (rope(q) * (1.0 / np.sqrt(DH))).astype(jnp.bfloat16)
    k_ref[...] = rope(k).astype(jnp.bfloat16)
    v_ref[...] = v.astype(jnp.bfloat16)


def _attn_body(q_ref, k_ref, v_ref, m_ref, o_ref):
    mask = m_ref[...] > 0.5
    for sub in range(2):
        sl = slice(sub * DH, (sub + 1) * DH)
        att = jax.lax.dot_general(
            q_ref[:, sl], k_ref[:, sl], (((1,), (1,)), ((), ())),
            preferred_element_type=jnp.float32)
        att = jnp.where(mask, att, -1e30)
        mx = jnp.max(att, axis=-1, keepdims=True)
        e = jnp.exp(att - mx)
        den = jnp.sum(e, axis=-1, keepdims=True)
        o = jnp.dot(e.astype(jnp.bfloat16), v_ref[:, sl],
                    preferred_element_type=jnp.float32)
        o_ref[:, sl] = (o / den).astype(jnp.bfloat16)


def _post_body(final_ref, ek_ref, o_ref, sw_ref, wo_ref, norm_ref,
               wg_ref, wu_ref, wd_ref, out_ref):
    x = final_ref[...] + ek_ref[...] + jnp.dot(
        o_ref[...], wo_ref[...], preferred_element_type=jnp.float32)
    h2 = x * jax.lax.rsqrt(jnp.mean(x * x, axis=-1, keepdims=True) + 1e-6)
    h2 = (h2 * norm_ref[...]).astype(jnp.bfloat16)
    g = jnp.dot(h2, wg_ref[...], preferred_element_type=jnp.float32)
    u = jnp.dot(h2, wu_ref[...], preferred_element_type=jnp.float32)
    act = ((g * jax.lax.logistic(g)) * u).astype(jnp.bfloat16)
    x = x + jnp.dot(act, wd_ref[...], preferred_element_type=jnp.float32)
    out_ref[...] = final_ref[...] + x * sw_ref[...]


def kernel(hidden_states, position_ids, expert_keys, params):
    Bb, Ss, Hh = hidden_states.shape
    flat = hidden_states.reshape(Ss, Hh)
    pos = position_ids.reshape(Ss, 1).astype(jnp.int32)

    kmask, sw = _pcall(
        _router_body,
        out_shape=(
            jax.ShapeDtypeStruct((NUM_EXPERTS, S), jnp.float32),
            jax.ShapeDtypeStruct((S, NUM_EXPERTS), jnp.float32),
        ),
    )(flat, expert_keys)

    nrow = S // ROW_BLK
    final = flat
    for d in range(NUM_REC):
        p = params[d]
        ek_row = expert_keys[1 << d][None, :]
        wq, wk, wv, wo, wg, wu, wd = (
            p[n].astype(jnp.bfloat16)
            for n in ("Wq", "Wk", "Wv", "Wo", "Wg", "Wu", "Wd"))

        q, k, v = _pcall(
            _qkv_body,
            grid=(nrow,),
            in_specs=[
                pl.BlockSpec((ROW_BLK, H), lambda i: (i, 0)),
                pl.BlockSpec((1, H), lambda i: (0, 0)),
                pl.BlockSpec((1, H), lambda i: (0, 0)),
                pl.BlockSpec((ROW_BLK, 1), lambda i: (i, 0)),
                pl.BlockSpec((H, H), lambda i: (0, 0)),
                pl.BlockSpec((H, H), lambda i: (0, 0)),
                pl.BlockSpec((H, H), lambda i: (0, 0)),
            ],
            out_specs=[
                pl.BlockSpec((ROW_BLK, H), lambda i: (i, 0)),
                pl.BlockSpec((ROW_BLK, H), lambda i: (i, 0)),
                pl.BlockSpec((ROW_BLK, H), lambda i: (i, 0)),
            ],
            out_shape=[jax.ShapeDtypeStruct((S, H), jnp.bfloat16)] * 3,
        )(final, ek_row, p["attn_norm"][None, :], pos, wq, wk, wv)

        o = _pcall(
            _attn_body,
            grid=(NH // 2,),
            in_specs=[
                pl.BlockSpec((S, 2 * DH), lambda h: (0, h)),
                pl.BlockSpec((S, 2 * DH), lambda h: (0, h)),
                pl.BlockSpec((S, 2 * DH), lambda h: (0, h)),
                pl.BlockSpec((1, S), lambda h: (0, 0)),
            ],
            out_specs=pl.BlockSpec((S, 2 * DH), lambda h: (0, h)),
            out_shape=jax.ShapeDtypeStruct((S, H), jnp.bfloat16),
        )(q, k, v, kmask[d][None, :])

        final = _pcall(
            _post_body,
            grid=(nrow,),
            in_specs=[
                pl.BlockSpec((ROW_BLK, H), lambda i: (i, 0)),
                pl.BlockSpec((1, H), lambda i: (0, 0)),
                pl.BlockSpec((ROW_BLK, H), lambda i: (i, 0)),
                pl.BlockSpec((ROW_BLK, 1), lambda i: (i, 0)),
                pl.BlockSpec((H, H), lambda i: (0, 0)),
                pl.BlockSpec((1, H), lambda i: (0, 0)),
                pl.BlockSpec((H, FF), lambda i: (0, 0)),
                pl.BlockSpec((H, FF), lambda i: (0, 0)),
                pl.BlockSpec((FF, H), lambda i: (0, 0)),
            ],
            out_specs=pl.BlockSpec((ROW_BLK, H), lambda i: (i, 0)),
            out_shape=jax.ShapeDtypeStruct((S, H), jnp.float32),
        )(final, ek_row, o, sw[:, d:d + 1], wo, p["mlp_norm"][None, :],
          wg, wu, wd)

    return final.reshape(Bb, Ss, Hh)


# speedup vs baseline: 1.1700x; 1.0894x over previous
import functools

import jax
import jax.numpy as jnp
import numpy as np
from jax import lax
from jax.experimental import pallas as pl
from jax.experimental.pallas import tpu as pltpu
from jax.experimental.pallas import tpu_sc as plsc

S, H = 2048, 1024
NH, DH = 16, 64
FF = 2048
NUM_REC = 3
NUM_EXPERTS = 2 ** NUM_REC
RB = 256
QB = 256
NW = 32
RPW = S // NW


def _pcall(body, **kw):
    return pl.pallas_call(body, **kw)


def _router_body(x_ref, ek_ref, sw_ref, dest_ref, cnt_ref):
    x = x_ref[...]
    scores = lax.dot_general(x, ek_ref[...], (((1,), (1,)), ((), ())),
                             preferred_element_type=jnp.float32)
    m = jnp.max(scores, axis=-1, keepdims=True)
    w = 1.0 / jnp.sum(jnp.exp(scores - m), axis=-1, keepdims=True)
    chosen = jnp.argmax(scores, axis=-1).astype(jnp.int32)
    bits = lax.broadcasted_iota(jnp.int32, (S, NUM_EXPERTS), 1)
    active = ((chosen[:, None] >> bits) & 1).astype(jnp.float32)
    sw_ref[...] = active * w

    r_io = lax.broadcasted_iota(jnp.int32, (S, S), 0)
    c_io = lax.broadcasted_iota(jnp.int32, (S, S), 1)
    tri = (r_io >= c_io).astype(jnp.bfloat16)
    rank = lax.dot_general(tri, active.astype(jnp.bfloat16),
                           (((1,), (0,)), ((), ())),
                           preferred_element_type=jnp.float32)
    total = rank[S - 1:S, :]
    t_col = lax.broadcasted_iota(jnp.int32, (S, 1), 0).astype(jnp.float32)
    dest = jnp.where(active > 0.5, rank - 1.0, total + t_col - rank)
    dest_ref[...] = dest.astype(jnp.int32)
    cnt_ref[...] = total.astype(jnp.int32)


def _sc_scatter(src, idx):
    mesh = plsc.VectorSubcoreMesh(core_axis_name="c", subcore_axis_name="s")

    @functools.partial(
        pl.kernel, mesh=mesh,
        out_type=jax.ShapeDtypeStruct((S, H), jnp.float32),
        scratch_types=[
            pltpu.VMEM((RPW,), jnp.int32),
            pltpu.VMEM((RPW, H), jnp.float32),
            pltpu.SemaphoreType.DMA,
        ],
    )
    def k(src_hbm, idx_hbm, out_hbm, idx_v, rows_v, sem):
        wid = lax.axis_index("s") * 2 + lax.axis_index("c")
        base = wid * RPW
        pltpu.sync_copy(idx_hbm.at[pl.ds(base, RPW)], idx_v)
        pltpu.sync_copy(src_hbm.at[pl.ds(base, RPW)], rows_v)
        pltpu.async_copy(rows_v, out_hbm.at[idx_v], sem).wait()

    return k(src, idx)


def _sc_gather(src, idx):
    mesh = plsc.VectorSubcoreMesh(core_axis_name="c", subcore_axis_name="s")

    @functools.partial(
        pl.kernel, mesh=mesh,
        out_type=jax.ShapeDtypeStruct((S, H), jnp.float32),
        scratch_types=[
            pltpu.VMEM((RPW,), jnp.int32),
            pltpu.VMEM((RPW, H), jnp.float32),
            pltpu.SemaphoreType.DMA,
        ],
    )
    def k(src_hbm, idx_hbm, out_hbm, idx_v, rows_v, sem):
        wid = lax.axis_index("s") * 2 + lax.axis_index("c")
        base = wid * RPW
        pltpu.sync_copy(idx_hbm.at[pl.ds(base, RPW)], idx_v)
        pltpu.async_copy(src_hbm.at[idx_v], rows_v, sem).wait()
        pltpu.sync_copy(rows_v, out_hbm.at[pl.ds(base, RPW)])

    return k(src, idx)


def _qkv_body(depth, cnt_ref, x_ref, ek_ref, norm_ref, dest_ref, pos_ref,
              wq_ref, wk_ref, wv_ref, q_ref, k_ref, v_ref):
    i = pl.program_id(0)
    a = cnt_ref[depth]

    @pl.when(i * RB < a)
    def _():
        x = x_ref[...] + ek_ref[...]
        h = x * lax.rsqrt(jnp.mean(x * x, axis=-1, keepdims=True) + 1e-6)
        h = (h * norm_ref[...]).astype(jnp.bfloat16)
        q = jnp.dot(h, wq_ref[...], preferred_element_type=jnp.float32)
        k = jnp.dot(h, wk_ref[...], preferred_element_type=jnp.float32)
        v = jnp.dot(h, wv_ref[...], preferred_element_type=jnp.float32)

        p_ids = i * RB + lax.broadcasted_iota(jnp.int32, (1, RB), 1)
        pmat = (dest_ref[...] == p_ids).astype(jnp.float32)
        posf = pos_ref[...].astype(jnp.float32)
        pos_blk = lax.dot_general(pmat, posf, (((0,), (0,)), ((), ())),
                                  preferred_element_type=jnp.float32)

        col = lax.broadcasted_iota(jnp.int32, (1, H), 1)
        offs = col % DH
        f = (offs % (DH // 2)).astype(jnp.float32)
        inv = jnp.exp(f * (-np.log(10000.0) / (DH // 2)))
        ang = pos_blk * inv
        cosf = jnp.cos(ang)
        sinf = jnp.sin(ang)
        first_half = offs < (DH // 2)

        def rope(t):
            rot_m = jnp.concatenate([t[:, DH // 2:], t[:, :DH // 2]], axis=1)
            rot_p = jnp.concatenate([t[:, -(DH // 2):], t[:, :-(DH // 2)]],
                                    axis=1)
            rot = jnp.where(first_half, -rot_m, rot_p)
            return t * cosf + rot * sinf

        q_ref[...] = (rope(q) * (1.0 / np.sqrt(DH))).astype(jnp.bfloat16)
        k_ref[...] = rope(k).astype(jnp.bfloat16)
        v_ref[...] = v.astype(jnp.bfloat16)

    @pl.when(i * RB >= a)
    def _():
        q_ref[...] = jnp.zeros_like(q_ref)
        k_ref[...] = jnp.zeros_like(k_ref)
        v_ref[...] = jnp.zeros_like(v_ref)


def _attn_body(depth, cnt_ref, q_ref, k_ref, v_ref, o_ref):
    qb = pl.program_id(1)
    a = cnt_ref[depth]

    @pl.when(qb * QB < a)
    def _():
        key_ok = lax.broadcasted_iota(jnp.int32, (1, S), 1) < a
        for sub in range(2):
            sl = slice(sub * DH, (sub + 1) * DH)
            att = lax.dot_general(
                q_ref[:, sl], k_ref[:, sl], (((1,), (1,)), ((), ())),
                preferred_element_type=jnp.float32)
            att = jnp.where(key_ok, att, -1e30)
            mx = jnp.max(att, axis=-1, keepdims=True)
            e = jnp.exp(att - mx)
            den = jnp.sum(e, axis=-1, keepdims=True)
            o = jnp.dot(e.astype(jnp.bfloat16), v_ref[:, sl],
                        preferred_element_type=jnp.float32)
            o_ref[:, sl] = (o / den).astype(jnp.bfloat16)

    @pl.when(qb * QB >= a)
    def _():
        o_ref[...] = jnp.zeros_like(o_ref)


def _post_body(depth, cnt_ref, x_ref, ek_ref, o_ref, wo_ref, norm_ref,
               wg_ref, wu_ref, wd_ref, y_ref):
    i = pl.program_id(0)
    a = cnt_ref[depth]

    @pl.when(i * RB < a)
    def _():
        x = x_ref[...] + ek_ref[...] + jnp.dot(
            o_ref[...], wo_ref[...], preferred_element_type=jnp.float32)
        h2 = x * lax.rsqrt(jnp.mean(x * x, axis=-1, keepdims=True) + 1e-6)
        h2 = (h2 * norm_ref[...]).astype(jnp.bfloat16)
        g = jnp.dot(h2, wg_ref[...], preferred_element_type=jnp.float32)
        u = jnp.dot(h2, wu_ref[...], preferred_element_type=jnp.float32)
        act = ((g * lax.logistic(g)) * u).astype(jnp.bfloat16)
        y_ref[...] = x + jnp.dot(act, wd_ref[...],
                                 preferred_element_type=jnp.float32)

    @pl.when(i * RB >= a)
    def _():
        y_ref[...] = jnp.zeros_like(y_ref)


def _combine_body(final_ref, y_ref, sw_ref, out_ref):
    out_ref[...] = final_ref[...] + y_ref[...] * sw_ref[...]


def kernel(hidden_states, position_ids, expert_keys, params):
    Bb, Ss, Hh = hidden_states.shape
    flat = hidden_states.reshape(Ss, Hh)
    pos = position_ids.reshape(Ss, 1).astype(jnp.int32)

    sw, dest3, cnt = _pcall(
        _router_body,
        out_shape=(
            jax.ShapeDtypeStruct((S, NUM_EXPERTS), jnp.float32),
            jax.ShapeDtypeStruct((S, NUM_EXPERTS), jnp.int32),
            jax.ShapeDtypeStruct((1, NUM_EXPERTS), jnp.int32),
        ),
    )(flat, expert_keys)
    cnt_flat = cnt.reshape(NUM_EXPERTS)

    final = flat
    for d in range(NUM_REC):
        p = params[d]
        ek_row = expert_keys[1 << d][None, :]
        wq, wk, wv, wo, wg, wu, wd = (
            p[n].astype(jnp.bfloat16)
            for n in ("Wq", "Wk", "Wv", "Wo", "Wg", "Wu", "Wd"))
        dest_d = dest3[:, d]
        dest_col = dest3[:, d:d + 1]

        xpack = _sc_scatter(final, dest_d)

        q, k, v = pl.pallas_call(
            functools.partial(_qkv_body, d),
            grid_spec=pltpu.PrefetchScalarGridSpec(
                num_scalar_prefetch=1,
                grid=(S // RB,),
                in_specs=[
                    pl.BlockSpec((RB, H), lambda i, c: (i, 0)),
                    pl.BlockSpec((1, H), lambda i, c: (0, 0)),
                    pl.BlockSpec((1, H), lambda i, c: (0, 0)),
                    pl.BlockSpec((S, 1), lambda i, c: (0, 0)),
                    pl.BlockSpec((S, 1), lambda i, c: (0, 0)),
                    pl.BlockSpec((H, H), lambda i, c: (0, 0)),
                    pl.BlockSpec((H, H), lambda i, c: (0, 0)),
                    pl.BlockSpec((H, H), lambda i, c: (0, 0)),
                ],
                out_specs=[
                    pl.BlockSpec((RB, H), lambda i, c: (i, 0)),
                    pl.BlockSpec((RB, H), lambda i, c: (i, 0)),
                    pl.BlockSpec((RB, H), lambda i, c: (i, 0)),
                ],
            ),
            out_shape=[jax.ShapeDtypeStruct((S, H), jnp.bfloat16)] * 3,
        )(cnt_flat, xpack, ek_row, p["attn_norm"][None, :], dest_col, pos,
          wq, wk, wv)

        o = pl.pallas_call(
            functools.partial(_attn_body, d),
            grid_spec=pltpu.PrefetchScalarGridSpec(
                num_scalar_prefetch=1,
                grid=(NH // 2, S // QB),
                in_specs=[
                    pl.BlockSpec((QB, 2 * DH), lambda h, qb, c: (qb, h)),
                    pl.BlockSpec((S, 2 * DH), lambda h, qb, c: (0, h)),
                    pl.BlockSpec((S, 2 * DH), lambda h, qb, c: (0, h)),
                ],
                out_specs=pl.BlockSpec((QB, 2 * DH), lambda h, qb, c: (qb, h)),
            ),
            out_shape=jax.ShapeDtypeStruct((S, H), jnp.bfloat16),
        )(cnt_flat, q, k, v)

        ypack = pl.pallas_call(
            functools.partial(_post_body, d),
            grid_spec=pltpu.PrefetchScalarGridSpec(
                num_scalar_prefetch=1,
                grid=(S // RB,),
                in_specs=[
                    pl.BlockSpec((RB, H), lambda i, c: (i, 0)),
                    pl.BlockSpec((1, H), lambda i, c: (0, 0)),
                    pl.BlockSpec((RB, H), lambda i, c: (i, 0)),
                    pl.BlockSpec((H, H), lambda i, c: (0, 0)),
                    pl.BlockSpec((1, H), lambda i, c: (0, 0)),
                    pl.BlockSpec((H, FF), lambda i, c: (0, 0)),
                    pl.BlockSpec((H, FF), lambda i, c: (0, 0)),
                    pl.BlockSpec((FF, H), lambda i, c: (0, 0)),
                ],
                out_specs=pl.BlockSpec((RB, H), lambda i, c: (i, 0)),
            ),
            out_shape=jax.ShapeDtypeStruct((S, H), jnp.float32),
        )(cnt_flat, xpack, ek_row, o, wo, p["mlp_norm"][None, :], wg, wu, wd)

        y = _sc_gather(ypack, dest_d)

        final = _pcall(
            _combine_body,
            grid=(S // 512,),
            in_specs=[
                pl.BlockSpec((512, H), lambda i: (i, 0)),
                pl.BlockSpec((512, H), lambda i: (i, 0)),
                pl.BlockSpec((512, 1), lambda i: (i, 0)),
            ],
            out_specs=pl.BlockSpec((512, H), lambda i: (i, 0)),
            out_shape=jax.ShapeDtypeStruct((S, H), jnp.float32),
        )(final, y, sw[:, d:d + 1])

    return final.reshape(Bb, Ss, Hh)
